# Initial kernel scaffold; baseline (speedup 1.0000x reference)
import jax
import jax.numpy as jnp
from jax.experimental import pallas as pl


def kernel(p, x, Wq, bq, Wk, bk, Wv, bv, Wp1, g_p1, b_p1, Wp2, bp2, g_a0, b_a0, Wa1, g_a1, b_a1, Wa2, ba2):
    raise NotImplementedError("write your pallas kernel here")


# speedup vs baseline: 15.7422x; 15.7422x over previous
import functools

import jax
import jax.numpy as jnp
from jax import lax
from jax.experimental import pallas as pl
from jax.experimental.pallas import tpu as pltpu
from jax.experimental.pallas import tpu_sc as plsc

_B = 4
_N = 4096
_C = 128
_K = 16
_EPS = 1e-5
_BN = _B * _N
_BNK = _BN * _K
_PB = 128
_RB = _PB * _K
_NSTEP = _BN // _PB
_NB1 = 512
_NB2 = 256
_INTERPRET = False


def _qkv_body(x_ref, w3_ref, b3_ref, q_ref, k_ref, v_ref):
    xb = x_ref[0]
    for j, out in enumerate((q_ref, k_ref, v_ref)):
        w = w3_ref[j]
        r = lax.dot_general(xb, w, (((0,), (1,)), ((), ())),
                            preferred_element_type=jnp.float32)
        out[0] = r + b3_ref[pl.ds(j, 1)]


def _qkv(x, W3, b3):
    grid = (_B, _N // _NB1)
    out = jax.ShapeDtypeStruct((_B, _N, _C), jnp.float32)
    q, k, v = pl.pallas_call(
        _qkv_body,
        grid=grid,
        in_specs=[
            pl.BlockSpec((1, _C, _NB1), lambda b, i: (b, 0, i)),
            pl.BlockSpec((3, _C, _C), lambda b, i: (0, 0, 0)),
            pl.BlockSpec((3, _C), lambda b, i: (0, 0)),
        ],
        out_specs=[pl.BlockSpec((1, _NB1, _C), lambda b, i: (b, i, 0))] * 3,
        out_shape=[out] * 3,
        interpret=_INTERPRET,
    )(x, W3, b3)
    return q, k, v


def _knn_body(pall_ref, prow_ref, idx_ref):
    b = pl.program_id(0)
    pa = pall_ref[0]
    pb = prow_ref[0]
    sqa = jnp.sum(pa * pa, axis=1)
    sqb = jnp.sum(pb * pb, axis=1)
    dot = lax.dot_general(pb, pa, (((1,), (1,)), ((), ())),
                          preferred_element_type=jnp.float32)
    d = sqb[:, None] + sqa[None, :] - 2.0 * dot
    d = jnp.maximum(d, 0.0)
    ik = lax.bitcast_convert_type(d, jnp.int32)
    lane = lax.broadcasted_iota(jnp.int32, (_NB2, _N), 1)
    ik = (ik & jnp.int32(-4096)) | lane
    big = jnp.int32(2**31 - 1)
    off = (b * _N).astype(jnp.int32)
    cols = []
    for _ in range(_K):
        m = jnp.min(ik, axis=1)
        cols.append((m & 4095) + off)
        ik = jnp.where(ik == m[:, None], big, ik)
    idx_ref[0] = jnp.stack(cols, axis=1)


def _knn(pp3):
    grid = (_B, _N // _NB2)
    return pl.pallas_call(
        _knn_body,
        grid=grid,
        in_specs=[
            pl.BlockSpec((1, _N, 16), lambda b, i: (b, 0, 0)),
            pl.BlockSpec((1, _NB2, 16), lambda b, i: (b, i, 0)),
        ],
        out_specs=pl.BlockSpec((1, _NB2, _K), lambda b, i: (b, i, 0)),
        out_shape=jax.ShapeDtypeStruct((_B, _N, _K), jnp.int32),
        interpret=_INTERPRET,
    )(pp3, pp3)


_NW = 32
_CHROWS = (_BNK // 128) // _NW


def _sc_gather_body(kt_ref, vt_ref, pp_ref, idx_ref,
                    nk_ref, nv_ref, gp_ref,
                    idxv, bk, bv, bp, semk, semv, semp):
    info = plsc.get_sparse_core_info()
    nc = info.num_cores
    wid = lax.axis_index("s") * nc + lax.axis_index("c")

    def step(t, carry):
        r = wid * _CHROWS + t
        pltpu.sync_copy(idx_ref.at[r], idxv)
        ck = pltpu.async_copy(kt_ref.at[idxv], bk, semk)
        cv = pltpu.async_copy(vt_ref.at[idxv], bv, semv)
        cp = pltpu.async_copy(pp_ref.at[idxv], bp, semp)
        ck.wait()
        pltpu.sync_copy(bk, nk_ref.at[pl.ds(r * 128, 128)])
        cv.wait()
        pltpu.sync_copy(bv, nv_ref.at[pl.ds(r * 128, 128)])
        cp.wait()
        pltpu.sync_copy(bp, gp_ref.at[pl.ds(r * 128, 128)])
        return carry

    lax.fori_loop(0, _CHROWS, step, 0)


def _sc_gather(kt2, vt2, pp2, idx2):
    info = plsc.get_sparse_core_info()
    mesh = plsc.VectorSubcoreMesh(core_axis_name="c", subcore_axis_name="s",
                                  num_cores=info.num_cores)
    fn = pl.kernel(
        _sc_gather_body,
        compiler_params=pltpu.CompilerParams(use_tc_tiling_on_sc=False),
        out_type=[
            jax.ShapeDtypeStruct((_BNK, _C), jnp.float32),
            jax.ShapeDtypeStruct((_BNK, _C), jnp.float32),
            jax.ShapeDtypeStruct((_BNK, 16), jnp.float32),
        ],
        mesh=mesh,
        scratch_types=[
            pltpu.VMEM((128,), jnp.int32),
            pltpu.VMEM((128, _C), jnp.float32),
            pltpu.VMEM((128, _C), jnp.float32),
            pltpu.VMEM((128, 16), jnp.float32),
            pltpu.SemaphoreType.DMA,
            pltpu.SemaphoreType.DMA,
            pltpu.SemaphoreType.DMA,
        ],
    )
    return fn(kt2, vt2, pp2, idx2)


def _stats1_body(gp_ref, pp_ref, w1_ref, acc_ref):
    @pl.when(pl.program_id(0) == 0)
    def _():
        acc_ref[...] = jnp.zeros_like(acc_ref)

    gpb = gp_ref[...]
    pb = pp_ref[...]
    rel = (gpb.reshape(_PB, _K, 16) - pb[:, None, :]).reshape(_RB, 16)
    h = lax.dot_general(rel, w1_ref[...], (((1,), (1,)), ((), ())),
                        preferred_element_type=jnp.float32)
    s = jnp.sum(h, axis=0)
    ss = jnp.sum(h * h, axis=0)
    z = jnp.zeros_like(s)
    acc_ref[...] += jnp.stack([s, ss, z, z, z, z, z, z])


def _stats1(gp2, pp2, W1p):
    return pl.pallas_call(
        _stats1_body,
        grid=(_NSTEP,),
        in_specs=[
            pl.BlockSpec((_RB, 16), lambda i: (i, 0)),
            pl.BlockSpec((_PB, 16), lambda i: (i, 0)),
            pl.BlockSpec((16, 16), lambda i: (0, 0)),
        ],
        out_specs=pl.BlockSpec((8, 16), lambda i: (0, 0)),
        out_shape=jax.ShapeDtypeStruct((8, 16), jnp.float32),
        interpret=_INTERPRET,
    )(gp2, pp2, W1p)


def _apre(gpb, pb, qb, nkb, w1, ab1, w2, bp2r):
    rel = (gpb.reshape(_PB, _K, 16) - pb[:, None, :]).reshape(_RB, 16)
    h = lax.dot_general(rel, w1, (((1,), (1,)), ((), ())),
                        preferred_element_type=jnp.float32)
    hp = jnp.maximum(h * ab1[0:1] + ab1[1:2], 0.0)
    nr = lax.dot_general(hp, w2, (((1,), (1,)), ((), ())),
                         preferred_element_type=jnp.float32) + bp2r
    q3 = jnp.broadcast_to(qb[:, None, :], (_PB, _K, _C)).reshape(_RB, _C)
    return q3 - nkb + nr, nr


def _stats2_body(gp_ref, pp_ref, q_ref, nk_ref, w1_ref, ab1_ref, w2_ref,
                 bp2_ref, acc_ref):
    @pl.when(pl.program_id(0) == 0)
    def _():
        acc_ref[...] = jnp.zeros_like(acc_ref)

    a_pre, _ = _apre(gp_ref[...], pp_ref[...], q_ref[...], nk_ref[...],
                     w1_ref[...], ab1_ref[...], w2_ref[...], bp2_ref[...])
    s = jnp.sum(a_pre, axis=0)
    ss = jnp.sum(a_pre * a_pre, axis=0)
    z = jnp.zeros_like(s)
    acc_ref[...] += jnp.stack([s, ss, z, z, z, z, z, z])


def _stats3_body(gp_ref, pp_ref, q_ref, nk_ref, w1_ref, ab1_ref, w2_ref,
                 bp2_ref, ab2_ref, wa1_ref, acc_ref):
    @pl.when(pl.program_id(0) == 0)
    def _():
        acc_ref[...] = jnp.zeros_like(acc_ref)

    a_pre, _ = _apre(gp_ref[...], pp_ref[...], q_ref[...], nk_ref[...],
                     w1_ref[...], ab1_ref[...], w2_ref[...], bp2_ref[...])
    act = jnp.maximum(a_pre * ab2_ref[0:1] + ab2_ref[1:2], 0.0)
    a1 = lax.dot_general(act, wa1_ref[...], (((1,), (1,)), ((), ())),
                         preferred_element_type=jnp.float32)
    s = jnp.sum(a1, axis=0)
    ss = jnp.sum(a1 * a1, axis=0)
    z = jnp.zeros_like(s)
    acc_ref[...] += jnp.stack([s, ss, z, z, z, z, z, z])


def _fsum(x3):
    r = x3
    for w in (8, 4, 2, 1):
        r = r[:, :w, :] + r[:, w:2 * w, :]
    return r


def _fmax(x3):
    r = x3
    for w in (8, 4, 2, 1):
        r = jnp.maximum(r[:, :w, :], r[:, w:2 * w, :])
    return r


def _final_body(gp_ref, pp_ref, q_ref, nk_ref, nv_ref, w1_ref, ab1_ref,
                w2_ref, bp2_ref, ab2_ref, wa1_ref, ab3_ref, wa2_ref,
                ba2_ref, y_ref):
    a_pre, nr = _apre(gp_ref[...], pp_ref[...], q_ref[...], nk_ref[...],
                      w1_ref[...], ab1_ref[...], w2_ref[...], bp2_ref[...])
    act = jnp.maximum(a_pre * ab2_ref[0:1] + ab2_ref[1:2], 0.0)
    a1 = lax.dot_general(act, wa1_ref[...], (((1,), (1,)), ((), ())),
                         preferred_element_type=jnp.float32)
    act2 = jnp.maximum(a1 * ab3_ref[0:1] + ab3_ref[1:2], 0.0)
    a2 = lax.dot_general(act2, wa2_ref[...], (((1,), (1,)), ((), ())),
                         preferred_element_type=jnp.float32) + ba2_ref[...]
    a3 = a2.reshape(_PB, _K, _C)
    m = _fmax(a3)
    e = jnp.exp(a3 - m)
    ssum = _fsum(e)
    att = e / ssum
    nv3 = nv_ref[...].reshape(_PB, _K, _C) + nr.reshape(_PB, _K, _C)
    y = _fsum(nv3 * att).reshape(_PB, _C)
    y_ref[0] = y.T


def _final(gp2, pp2, qt2, nk2, nv2, W1p, ab1, W2p, bp2r, ab2, Wa1, ab3,
           Wa2, ba2r):
    nblk = _N // _PB
    return pl.pallas_call(
        _final_body,
        grid=(_NSTEP,),
        in_specs=[
            pl.BlockSpec((_RB, 16), lambda i: (i, 0)),
            pl.BlockSpec((_PB, 16), lambda i: (i, 0)),
            pl.BlockSpec((_PB, _C), lambda i: (i, 0)),
            pl.BlockSpec((_RB, _C), lambda i: (i, 0)),
            pl.BlockSpec((_RB, _C), lambda i: (i, 0)),
            pl.BlockSpec((16, 16), lambda i: (0, 0)),
            pl.BlockSpec((2, 16), lambda i: (0, 0)),
            pl.BlockSpec((_C, 16), lambda i: (0, 0)),
            pl.BlockSpec((1, _C), lambda i: (0, 0)),
            pl.BlockSpec((2, _C), lambda i: (0, 0)),
            pl.BlockSpec((_C, _C), lambda i: (0, 0)),
            pl.BlockSpec((2, _C), lambda i: (0, 0)),
            pl.BlockSpec((_C, _C), lambda i: (0, 0)),
            pl.BlockSpec((1, _C), lambda i: (0, 0)),
        ],
        out_specs=pl.BlockSpec((1, _C, _PB),
                               lambda i: (i // nblk, 0, i % nblk)),
        out_shape=jax.ShapeDtypeStruct((_B, _C, _N), jnp.float32),
        interpret=_INTERPRET,
    )(gp2, pp2, qt2, nk2, nv2, W1p, ab1, W2p, bp2r, ab2, Wa1, ab3, Wa2, ba2r)


def _stats2(gp2, pp2, qt2, nk2, W1p, ab1, W2p, bp2r):
    return pl.pallas_call(
        _stats2_body,
        grid=(_NSTEP,),
        in_specs=[
            pl.BlockSpec((_RB, 16), lambda i: (i, 0)),
            pl.BlockSpec((_PB, 16), lambda i: (i, 0)),
            pl.BlockSpec((_PB, _C), lambda i: (i, 0)),
            pl.BlockSpec((_RB, _C), lambda i: (i, 0)),
            pl.BlockSpec((16, 16), lambda i: (0, 0)),
            pl.BlockSpec((2, 16), lambda i: (0, 0)),
            pl.BlockSpec((_C, 16), lambda i: (0, 0)),
            pl.BlockSpec((1, _C), lambda i: (0, 0)),
        ],
        out_specs=pl.BlockSpec((8, _C), lambda i: (0, 0)),
        out_shape=jax.ShapeDtypeStruct((8, _C), jnp.float32),
        interpret=_INTERPRET,
    )(gp2, pp2, qt2, nk2, W1p, ab1, W2p, bp2r)


def _stats3(gp2, pp2, qt2, nk2, W1p, ab1, W2p, bp2r, ab2, Wa1):
    return pl.pallas_call(
        _stats3_body,
        grid=(_NSTEP,),
        in_specs=[
            pl.BlockSpec((_RB, 16), lambda i: (i, 0)),
            pl.BlockSpec((_PB, 16), lambda i: (i, 0)),
            pl.BlockSpec((_PB, _C), lambda i: (i, 0)),
            pl.BlockSpec((_RB, _C), lambda i: (i, 0)),
            pl.BlockSpec((16, 16), lambda i: (0, 0)),
            pl.BlockSpec((2, 16), lambda i: (0, 0)),
            pl.BlockSpec((_C, 16), lambda i: (0, 0)),
            pl.BlockSpec((1, _C), lambda i: (0, 0)),
            pl.BlockSpec((2, _C), lambda i: (0, 0)),
            pl.BlockSpec((_C, _C), lambda i: (0, 0)),
        ],
        out_specs=pl.BlockSpec((8, _C), lambda i: (0, 0)),
        out_shape=jax.ShapeDtypeStruct((8, _C), jnp.float32),
        interpret=_INTERPRET,
    )(gp2, pp2, qt2, nk2, W1p, ab1, W2p, bp2r, ab2, Wa1)


def _ab(s, ss, g, b):
    m = s / _BNK
    v = ss / _BNK - m * m
    a = g / jnp.sqrt(v + _EPS)
    return a, b - m * a


def kernel(p, x, Wq, bq, Wk, bk, Wv, bv, Wp1, g_p1, b_p1, Wp2, bp2,
           g_a0, b_a0, Wa1, g_a1, b_a1, Wa2, ba2):
    f32 = jnp.float32
    W3 = jnp.stack([Wq, Wk, Wv])
    b3 = jnp.stack([bq, bk, bv])
    qt, kt, vt = _qkv(x, W3, b3)
    qt2 = qt.reshape(_BN, _C)
    kt2 = kt.reshape(_BN, _C)
    vt2 = vt.reshape(_BN, _C)

    pp3 = jnp.concatenate(
        [p, jnp.zeros((_B, _N, 13), f32)], axis=-1)
    pp2 = pp3.reshape(_BN, 16)

    idx = _knn(pp3)
    idx2 = idx.reshape(_BNK // 128, 128)

    nk2, nv2, gp2 = _sc_gather(kt2, vt2, pp2, idx2)

    W1p = jnp.zeros((16, 16), f32).at[:3, :3].set(Wp1)
    acc1 = _stats1(gp2, pp2, W1p)
    a1v, b1v = _ab(acc1[0, :3], acc1[1, :3], g_p1, b_p1)
    z13 = jnp.zeros((13,), f32)
    ab1 = jnp.stack([jnp.concatenate([a1v, z13]),
                     jnp.concatenate([b1v, z13])])
    W2p = jnp.zeros((_C, 16), f32).at[:, :3].set(Wp2)
    bp2r = bp2.reshape(1, _C)

    acc2 = _stats2(gp2, pp2, qt2, nk2, W1p, ab1, W2p, bp2r)
    a2v, b2v = _ab(acc2[0], acc2[1], g_a0, b_a0)
    ab2 = jnp.stack([a2v, b2v])

    acc3 = _stats3(gp2, pp2, qt2, nk2, W1p, ab1, W2p, bp2r, ab2, Wa1)
    a3v, b3v = _ab(acc3[0], acc3[1], g_a1, b_a1)
    ab3 = jnp.stack([a3v, b3v])

    ba2r = ba2.reshape(1, _C)
    y = _final(gp2, pp2, qt2, nk2, nv2, W1p, ab1, W2p, bp2r, ab2, Wa1,
               ab3, Wa2, ba2r)
    return y
